# baseline (speedup 1.0000x reference)
import functools

import jax
import jax.numpy as jnp
from jax import lax
from jax.experimental import pallas as pl
from jax.experimental.pallas import tpu as pltpu
from jax.experimental.pallas import tpu_sc as plsc

_TEMP = 101.0

_NC = 2
_NS = 16
_NW = _NC * _NS
_BH = 80
_BA = 80
_HW = 16


def _mesh():
    return plsc.VectorSubcoreMesh(core_axis_name="c", subcore_axis_name="s")


def _make_hist(N, NJ):
    ZR = 128
    STRIDE, SZ = 624, 640
    assert STRIDE * (_NS - 1) + SZ == N and SZ % ZR == 0

    @functools.partial(
        pl.kernel,
        out_type=jax.ShapeDtypeStruct((_NC, N, _HW), jnp.float32),
        mesh=_mesh(),
        scratch_types=[
            pltpu.VMEM((NJ, _BH), jnp.int32),
            pltpu.VMEM((_BH, _HW), jnp.float32),
            pltpu.VMEM((ZR, _HW), jnp.float32),
            pltpu.VMEM_SHARED((N, _HW), jnp.float32),
            pltpu.SemaphoreType.DMA,
        ],
        compiler_params=pltpu.CompilerParams(use_tc_tiling_on_sc=False),
    )
    def hist(dst_hbm, out_hbm, dst_v, ones_v, z_v, acc, sh):
        c = lax.axis_index("c")
        s = lax.axis_index("s")
        wid = s * _NC + c

        def fill_ones(r, carry):
            ones_v[r, :] = jnp.full((_HW,), 1.0, jnp.float32)
            return carry

        lax.fori_loop(0, _BH, fill_ones, 0)

        def fill_zero(r, carry):
            z_v[r, :] = jnp.zeros((_HW,), jnp.float32)
            return carry

        lax.fori_loop(0, ZR, fill_zero, 0)

        def zcopy(k, carry):
            pltpu.sync_copy(z_v, acc.at[pl.ds(s * STRIDE + k * ZR, ZR)])
            return carry

        lax.fori_loop(0, SZ // ZR, zcopy, 0)
        plsc.subcore_barrier()

        pltpu.sync_copy(dst_hbm.at[wid], dst_v)

        HDEPTH = 6

        def scat(j, carry):
            pltpu.async_copy(ones_v, acc.at[dst_v.at[j]], sh, add=True)

            @pl.when(j >= HDEPTH)
            def _():
                pltpu.make_async_copy(ones_v, acc.at[dst_v.at[0]],
                                      sh).wait()
            return carry

        lax.fori_loop(0, NJ, scat, 0)
        for _ in range(min(HDEPTH, NJ)):
            pltpu.make_async_copy(ones_v, acc.at[dst_v.at[0]], sh).wait()
        plsc.subcore_barrier()
        pltpu.sync_copy(acc.at[pl.ds(s * STRIDE, SZ)],
                        out_hbm.at[c, pl.ds(s * STRIDE, SZ)])

    return hist


def _make_agg(N, D, NJ):
    ZR = 128
    STRIDE, SZ = 624, 640
    assert STRIDE * (_NS - 1) + SZ == N and SZ % ZR == 0
    DH = D // 2

    @functools.partial(
        pl.kernel,
        out_type=jax.ShapeDtypeStruct((_NC, N, DH), jnp.float32),
        mesh=_mesh(),
        scratch_types=[
            pltpu.VMEM((NJ, _BA), jnp.int32),
            pltpu.VMEM((NJ, _BA), jnp.int32),
            *([pltpu.VMEM((_BA, DH), jnp.float32)] * 8),
            pltpu.VMEM((ZR, DH), jnp.float32),
            pltpu.VMEM_SHARED((N, DH), jnp.float32),
            *([pltpu.SemaphoreType.DMA] * 16),
        ],
        compiler_params=pltpu.CompilerParams(use_tc_tiling_on_sc=False),
    )
    def agg(src_hbm, dst_hbm, xsa_hbm, xsb_hbm, out_hbm,
            src_v, dst_v, b0, b1, b2, b3, b4, b5, b6, b7, z_v, acc,
            g0, g1, g2, g3, g4, g5, g6, g7,
            t0, t1, t2, t3, t4, t5, t6, t7):
        bufs = (b0, b1, b2, b3, b4, b5, b6, b7)
        sg = (g0, g1, g2, g3, g4, g5, g6, g7)
        ss = (t0, t1, t2, t3, t4, t5, t6, t7)
        c = lax.axis_index("c")
        s = lax.axis_index("s")

        def fill_zero(r, carry):
            for q in range(DH // 16):
                z_v[r, pl.ds(q * 16, 16)] = jnp.zeros((16,), jnp.float32)
            return carry

        lax.fori_loop(0, ZR, fill_zero, 0)

        def zcopy(k, carry):
            pltpu.sync_copy(z_v, acc.at[pl.ds(s * STRIDE + k * ZR, ZR)])
            return carry

        lax.fori_loop(0, SZ // ZR, zcopy, 0)
        plsc.subcore_barrier()

        pltpu.sync_copy(src_hbm.at[s], src_v)
        pltpu.sync_copy(dst_hbm.at[s], dst_v)

        NB = 8
        GD = 7
        SD = 1
        assert NJ % NB in (0, 2) and NJ >= 2 * NB

        def run(xs_hbm):
            def gather(j, b):
                pltpu.async_copy(xs_hbm.at[src_v.at[j]], bufs[b], sg[b])

            def wait_gather(j, b):
                pltpu.make_async_copy(xs_hbm.at[src_v.at[j]],
                                      bufs[b], sg[b]).wait()

            def scatter(j, b):
                pltpu.async_copy(bufs[b], acc.at[dst_v.at[j]], ss[b],
                                 add=True)

            def drain_scatter(b):
                pltpu.make_async_copy(bufs[b], acc.at[dst_v.at[0]],
                                      ss[b]).wait()

            for j in range(GD):
                gather(j, j)

            def octet(i, carry):
                for b in range(NB):
                    j = NB * i + b
                    wait_gather(j, b)
                    if b < SD:
                        @pl.when(j >= SD)
                        def _():
                            drain_scatter((b + GD) % NB)
                    else:
                        drain_scatter((b + GD) % NB)

                    @pl.when(j + GD < NJ)
                    def _():
                        gather(j + GD, (b + GD) % NB)

                    scatter(j, b)
                return carry

            lax.fori_loop(0, NJ // NB, octet, 0)
            for j in range(NJ - (NJ % NB), NJ):
                b = j % NB
                wait_gather(j, b)
                drain_scatter((b + GD) % NB)
                scatter(j, b)
            for j in range(NJ - SD, NJ):
                drain_scatter(j % NB)

        @pl.when(c == 0)
        def _():
            run(xsa_hbm)

        @pl.when(c == 1)
        def _():
            run(xsb_hbm)

        plsc.subcore_barrier()
        pltpu.sync_copy(acc.at[pl.ds(s * STRIDE, SZ)],
                        out_hbm.at[c, pl.ds(s * STRIDE, SZ)])

    return agg


def _prescale_body(hist_ref, x_ref, xsa_ref, xsb_ref, dis_ref):
    deg = hist_ref[0, :, 0:1] + hist_ref[1, :, 0:1] + 1.0
    dis = lax.rsqrt(deg)
    dis_ref[...] = dis
    xs = x_ref[...] * dis
    dh = xsa_ref.shape[1]
    xsa_ref[...] = xs[:, :dh]
    xsb_ref[...] = xs[:, dh:]


def _prescale(hist, x, RB):
    N, D = x.shape
    DH = D // 2
    grid = (N // RB,)
    return pl.pallas_call(
        _prescale_body,
        grid=grid,
        in_specs=[
            pl.BlockSpec((_NC, RB, _HW), lambda i: (0, i, 0)),
            pl.BlockSpec((RB, D), lambda i: (i, 0)),
        ],
        out_specs=[
            pl.BlockSpec((RB, DH), lambda i: (i, 0)),
            pl.BlockSpec((RB, DH), lambda i: (i, 0)),
            pl.BlockSpec((RB, 1), lambda i: (i, 0)),
        ],
        out_shape=[
            jax.ShapeDtypeStruct((N, DH), jnp.float32),
            jax.ShapeDtypeStruct((N, DH), jnp.float32),
            jax.ShapeDtypeStruct((N, 1), jnp.float32),
        ],
    )(hist, x)


def _gates(top, G):
    logits = lax.dot_general(top, G, (((1,), (1,)), ((), ())),
                             preferred_element_type=jnp.float32) / _TEMP
    m = jnp.max(logits, axis=1, keepdims=True)
    e = jnp.exp(logits - m)
    return e / jnp.sum(e, axis=1, keepdims=True)


def _mixture(parts_ref, dis, xprev, top_ref, W_ref, b_ref, G_ref):
    aggf = jnp.concatenate([parts_ref[0], parts_ref[1]], axis=1)
    z = dis * aggf + (dis * dis) * xprev
    g = _gates(top_ref[...], G_ref[...])
    acc = jnp.zeros_like(z)
    for i in range(W_ref.shape[0]):
        eo = lax.dot_general(z, W_ref[i], (((1,), (1,)), ((), ())),
                             preferred_element_type=jnp.float32) + b_ref[i]
        acc = acc + g[:, i:i + 1] * jnp.maximum(eo, 0.0)
    return acc


def _mix1_body(parts_ref, dis_ref, x_ref, top_ref, W_ref, b_ref, G_ref,
               h_ref, xsa_ref, xsb_ref):
    dis = dis_ref[...]
    h = _mixture(parts_ref, dis, x_ref[...], top_ref, W_ref, b_ref, G_ref)
    h_ref[...] = h
    xs = h * dis
    dh = xsa_ref.shape[1]
    xsa_ref[...] = xs[:, :dh]
    xsb_ref[...] = xs[:, dh:]


def _mix2_body(parts_ref, dis_ref, h1_ref, top_ref, W_ref, b_ref, G_ref,
               fcW_ref, fcb_ref, out_ref):
    dis = dis_ref[...]
    h = _mixture(parts_ref, dis, h1_ref[...], top_ref, W_ref, b_ref, G_ref)
    out_ref[...] = lax.dot_general(h, fcW_ref[...], (((1,), (1,)), ((), ())),
                                   preferred_element_type=jnp.float32) \
        + fcb_ref[...]


def _mix_specs(N, D, TOP, EX, RB):
    return [
        pl.BlockSpec((_NC, RB, D // 2), lambda i: (0, i, 0)),
        pl.BlockSpec((RB, 1), lambda i: (i, 0)),
        pl.BlockSpec((RB, D), lambda i: (i, 0)),
        pl.BlockSpec((RB, TOP), lambda i: (i, 0)),
        pl.BlockSpec((EX, D, D), lambda i: (0, 0, 0)),
        pl.BlockSpec((EX, D), lambda i: (0, 0)),
        pl.BlockSpec((EX, TOP), lambda i: (0, 0)),
    ]


def _mix1(parts, dis, x, top, W, b, G, RB):
    N, D = x.shape
    EX, TOP = G.shape
    return pl.pallas_call(
        _mix1_body,
        grid=(N // RB,),
        in_specs=_mix_specs(N, D, TOP, EX, RB),
        out_specs=[
            pl.BlockSpec((RB, D), lambda i: (i, 0)),
            pl.BlockSpec((RB, D // 2), lambda i: (i, 0)),
            pl.BlockSpec((RB, D // 2), lambda i: (i, 0)),
        ],
        out_shape=[
            jax.ShapeDtypeStruct((N, D), jnp.float32),
            jax.ShapeDtypeStruct((N, D // 2), jnp.float32),
            jax.ShapeDtypeStruct((N, D // 2), jnp.float32),
        ],
    )(parts, dis, x, top, W, b, G)


def _mix2(parts, dis, h1, top, W, b, G, fcW, fcb, RB):
    N, D = h1.shape
    EX, TOP = G.shape
    specs = _mix_specs(N, D, TOP, EX, RB) + [
        pl.BlockSpec((D, D), lambda i: (0, 0)),
        pl.BlockSpec((D,), lambda i: (0,)),
    ]
    return pl.pallas_call(
        _mix2_body,
        grid=(N // RB,),
        in_specs=specs,
        out_specs=pl.BlockSpec((RB, D), lambda i: (i, 0)),
        out_shape=jax.ShapeDtypeStruct((N, D), jnp.float32),
    )(parts, dis, h1, top, W, b, G, fcW, fcb)


def kernel(x, edge_index, top_features, W1, b1, W2, b2, G1, G2, fcW, fcb):
    N, D = x.shape
    E = edge_index.shape[1]
    NJ32 = E // (_NW * _BH)
    RB = 2000

    NJ16 = E // (_NS * _BA)
    src16 = edge_index[0].reshape(_NS, NJ16, _BA)
    dst16 = edge_index[1].reshape(_NS, NJ16, _BA)
    dst32 = edge_index[1].reshape(_NW, NJ32, _BH)

    hist_k = _make_hist(N, NJ32)
    agg_k = _make_agg(N, D, NJ16)

    hist = hist_k(dst32)
    xs1a, xs1b, dis = _prescale(hist, x, RB)
    parts1 = agg_k(src16, dst16, xs1a, xs1b)
    h1, xs2a, xs2b = _mix1(parts1, dis, x, top_features, W1, b1, G1, RB)
    parts2 = agg_k(src16, dst16, xs2a, xs2b)
    return _mix2(parts2, dis, h1, top_features, W2, b2, G2, fcW, fcb, RB)


# speedup vs baseline: 1.0051x; 1.0051x over previous
import functools

import jax
import jax.numpy as jnp
from jax import lax
from jax.experimental import pallas as pl
from jax.experimental.pallas import tpu as pltpu
from jax.experimental.pallas import tpu_sc as plsc

_TEMP = 101.0

_NC = 2
_NS = 16
_NW = _NC * _NS
_BH = 80
_BA = 80
_HW = 16


def _mesh():
    return plsc.VectorSubcoreMesh(core_axis_name="c", subcore_axis_name="s")


def _make_hist(N, NJ):
    ZR = 128
    STRIDE, SZ = 624, 640
    assert STRIDE * (_NS - 1) + SZ == N and SZ % ZR == 0

    @functools.partial(
        pl.kernel,
        out_type=jax.ShapeDtypeStruct((_NC, N, _HW), jnp.float32),
        mesh=_mesh(),
        scratch_types=[
            pltpu.VMEM((NJ, _BH), jnp.int32),
            pltpu.VMEM((_BH, _HW), jnp.float32),
            pltpu.VMEM((ZR, _HW), jnp.float32),
            pltpu.VMEM_SHARED((N, _HW), jnp.float32),
            pltpu.SemaphoreType.DMA,
        ],
        compiler_params=pltpu.CompilerParams(use_tc_tiling_on_sc=False),
    )
    def hist(dst_hbm, out_hbm, dst_v, ones_v, z_v, acc, sh):
        c = lax.axis_index("c")
        s = lax.axis_index("s")
        wid = s * _NC + c

        def fill_ones(r, carry):
            ones_v[r, :] = jnp.full((_HW,), 1.0, jnp.float32)
            return carry

        lax.fori_loop(0, _BH, fill_ones, 0)

        def fill_zero(r, carry):
            z_v[r, :] = jnp.zeros((_HW,), jnp.float32)
            return carry

        lax.fori_loop(0, ZR, fill_zero, 0)

        def zcopy(k, carry):
            pltpu.sync_copy(z_v, acc.at[pl.ds(s * STRIDE + k * ZR, ZR)])
            return carry

        lax.fori_loop(0, SZ // ZR, zcopy, 0)
        plsc.subcore_barrier()

        pltpu.sync_copy(dst_hbm.at[wid], dst_v)

        HDEPTH = 6

        def scat(j, carry):
            pltpu.async_copy(ones_v, acc.at[dst_v.at[j]], sh, add=True)

            @pl.when(j >= HDEPTH)
            def _():
                pltpu.make_async_copy(ones_v, acc.at[dst_v.at[0]],
                                      sh).wait()
            return carry

        lax.fori_loop(0, NJ, scat, 0)
        for _ in range(min(HDEPTH, NJ)):
            pltpu.make_async_copy(ones_v, acc.at[dst_v.at[0]], sh).wait()
        plsc.subcore_barrier()
        pltpu.sync_copy(acc.at[pl.ds(s * STRIDE, SZ)],
                        out_hbm.at[c, pl.ds(s * STRIDE, SZ)])

    return hist


def _make_agg(N, D, NJ):
    ZR = 128
    STRIDE, SZ = 624, 640
    assert STRIDE * (_NS - 1) + SZ == N and SZ % ZR == 0
    DH = D // 2

    @functools.partial(
        pl.kernel,
        out_type=jax.ShapeDtypeStruct((_NC, N, DH), jnp.float32),
        mesh=_mesh(),
        scratch_types=[
            pltpu.VMEM((NJ, _BA), jnp.int32),
            pltpu.VMEM((NJ, _BA), jnp.int32),
            *([pltpu.VMEM((_BA, DH), jnp.float32)] * 8),
            pltpu.VMEM((ZR, DH), jnp.float32),
            pltpu.VMEM_SHARED((N, DH), jnp.float32),
            *([pltpu.SemaphoreType.DMA] * 16),
        ],
        compiler_params=pltpu.CompilerParams(use_tc_tiling_on_sc=False),
    )
    def agg(src_hbm, dst_hbm, xsa_hbm, xsb_hbm, out_hbm,
            src_v, dst_v, b0, b1, b2, b3, b4, b5, b6, b7, z_v, acc,
            g0, g1, g2, g3, g4, g5, g6, g7,
            t0, t1, t2, t3, t4, t5, t6, t7):
        bufs = (b0, b1, b2, b3, b4, b5, b6, b7)
        sg = (g0, g1, g2, g3, g4, g5, g6, g7)
        ss = (t0, t1, t2, t3, t4, t5, t6, t7)
        c = lax.axis_index("c")
        s = lax.axis_index("s")

        def fill_zero(r, carry):
            for q in range(DH // 16):
                z_v[r, pl.ds(q * 16, 16)] = jnp.zeros((16,), jnp.float32)
            return carry

        lax.fori_loop(0, ZR, fill_zero, 0)

        def zcopy(k, carry):
            pltpu.sync_copy(z_v, acc.at[pl.ds(s * STRIDE + k * ZR, ZR)])
            return carry

        lax.fori_loop(0, SZ // ZR, zcopy, 0)
        plsc.subcore_barrier()

        pltpu.sync_copy(src_hbm.at[s], src_v)
        pltpu.sync_copy(dst_hbm.at[s], dst_v)

        NB = 8
        GD = 6
        SD = 2
        assert NJ % NB in (0, 2) and NJ >= 2 * NB

        def run(xs_hbm):
            def gather(j, b):
                pltpu.async_copy(xs_hbm.at[src_v.at[j]], bufs[b], sg[b])

            def wait_gather(j, b):
                pltpu.make_async_copy(xs_hbm.at[src_v.at[j]],
                                      bufs[b], sg[b]).wait()

            def scatter(j, b):
                pltpu.async_copy(bufs[b], acc.at[dst_v.at[j]], ss[b],
                                 add=True)

            def drain_scatter(b):
                pltpu.make_async_copy(bufs[b], acc.at[dst_v.at[0]],
                                      ss[b]).wait()

            for j in range(GD):
                gather(j, j)

            def octet(i, carry):
                for b in range(NB):
                    j = NB * i + b
                    wait_gather(j, b)
                    if b < SD:
                        @pl.when(j >= SD)
                        def _():
                            drain_scatter((b + GD) % NB)
                    else:
                        drain_scatter((b + GD) % NB)

                    @pl.when(j + GD < NJ)
                    def _():
                        gather(j + GD, (b + GD) % NB)

                    scatter(j, b)
                return carry

            lax.fori_loop(0, NJ // NB, octet, 0)
            for j in range(NJ - (NJ % NB), NJ):
                b = j % NB
                wait_gather(j, b)
                drain_scatter((b + GD) % NB)
                scatter(j, b)
            for j in range(NJ - SD, NJ):
                drain_scatter(j % NB)

        @pl.when(c == 0)
        def _():
            run(xsa_hbm)

        @pl.when(c == 1)
        def _():
            run(xsb_hbm)

        plsc.subcore_barrier()
        pltpu.sync_copy(acc.at[pl.ds(s * STRIDE, SZ)],
                        out_hbm.at[c, pl.ds(s * STRIDE, SZ)])

    return agg


def _prescale_body(hist_ref, x_ref, xsa_ref, xsb_ref, dis_ref):
    deg = hist_ref[0, :, 0:1] + hist_ref[1, :, 0:1] + 1.0
    dis = lax.rsqrt(deg)
    dis_ref[...] = dis
    xs = x_ref[...] * dis
    dh = xsa_ref.shape[1]
    xsa_ref[...] = xs[:, :dh]
    xsb_ref[...] = xs[:, dh:]


def _prescale(hist, x, RB):
    N, D = x.shape
    DH = D // 2
    grid = (N // RB,)
    return pl.pallas_call(
        _prescale_body,
        grid=grid,
        in_specs=[
            pl.BlockSpec((_NC, RB, _HW), lambda i: (0, i, 0)),
            pl.BlockSpec((RB, D), lambda i: (i, 0)),
        ],
        out_specs=[
            pl.BlockSpec((RB, DH), lambda i: (i, 0)),
            pl.BlockSpec((RB, DH), lambda i: (i, 0)),
            pl.BlockSpec((RB, 1), lambda i: (i, 0)),
        ],
        out_shape=[
            jax.ShapeDtypeStruct((N, DH), jnp.float32),
            jax.ShapeDtypeStruct((N, DH), jnp.float32),
            jax.ShapeDtypeStruct((N, 1), jnp.float32),
        ],
    )(hist, x)


def _gates(top, G):
    logits = lax.dot_general(top, G, (((1,), (1,)), ((), ())),
                             preferred_element_type=jnp.float32) / _TEMP
    m = jnp.max(logits, axis=1, keepdims=True)
    e = jnp.exp(logits - m)
    return e / jnp.sum(e, axis=1, keepdims=True)


def _mixture(parts_ref, dis, xprev, top_ref, W_ref, b_ref, G_ref):
    aggf = jnp.concatenate([parts_ref[0], parts_ref[1]], axis=1)
    z = dis * aggf + (dis * dis) * xprev
    g = _gates(top_ref[...], G_ref[...])
    acc = jnp.zeros_like(z)
    for i in range(W_ref.shape[0]):
        eo = lax.dot_general(z, W_ref[i], (((1,), (1,)), ((), ())),
                             preferred_element_type=jnp.float32) + b_ref[i]
        acc = acc + g[:, i:i + 1] * jnp.maximum(eo, 0.0)
    return acc


def _mix1_body(parts_ref, dis_ref, x_ref, top_ref, W_ref, b_ref, G_ref,
               h_ref, xsa_ref, xsb_ref):
    dis = dis_ref[...]
    h = _mixture(parts_ref, dis, x_ref[...], top_ref, W_ref, b_ref, G_ref)
    h_ref[...] = h
    xs = h * dis
    dh = xsa_ref.shape[1]
    xsa_ref[...] = xs[:, :dh]
    xsb_ref[...] = xs[:, dh:]


def _mix2_body(parts_ref, dis_ref, h1_ref, top_ref, W_ref, b_ref, G_ref,
               fcW_ref, fcb_ref, out_ref):
    dis = dis_ref[...]
    h = _mixture(parts_ref, dis, h1_ref[...], top_ref, W_ref, b_ref, G_ref)
    out_ref[...] = lax.dot_general(h, fcW_ref[...], (((1,), (1,)), ((), ())),
                                   preferred_element_type=jnp.float32) \
        + fcb_ref[...]


def _mix_specs(N, D, TOP, EX, RB):
    return [
        pl.BlockSpec((_NC, RB, D // 2), lambda i: (0, i, 0)),
        pl.BlockSpec((RB, 1), lambda i: (i, 0)),
        pl.BlockSpec((RB, D), lambda i: (i, 0)),
        pl.BlockSpec((RB, TOP), lambda i: (i, 0)),
        pl.BlockSpec((EX, D, D), lambda i: (0, 0, 0)),
        pl.BlockSpec((EX, D), lambda i: (0, 0)),
        pl.BlockSpec((EX, TOP), lambda i: (0, 0)),
    ]


def _mix1(parts, dis, x, top, W, b, G, RB):
    N, D = x.shape
    EX, TOP = G.shape
    return pl.pallas_call(
        _mix1_body,
        grid=(N // RB,),
        in_specs=_mix_specs(N, D, TOP, EX, RB),
        out_specs=[
            pl.BlockSpec((RB, D), lambda i: (i, 0)),
            pl.BlockSpec((RB, D // 2), lambda i: (i, 0)),
            pl.BlockSpec((RB, D // 2), lambda i: (i, 0)),
        ],
        out_shape=[
            jax.ShapeDtypeStruct((N, D), jnp.float32),
            jax.ShapeDtypeStruct((N, D // 2), jnp.float32),
            jax.ShapeDtypeStruct((N, D // 2), jnp.float32),
        ],
    )(parts, dis, x, top, W, b, G)


def _mix2(parts, dis, h1, top, W, b, G, fcW, fcb, RB):
    N, D = h1.shape
    EX, TOP = G.shape
    specs = _mix_specs(N, D, TOP, EX, RB) + [
        pl.BlockSpec((D, D), lambda i: (0, 0)),
        pl.BlockSpec((D,), lambda i: (0,)),
    ]
    return pl.pallas_call(
        _mix2_body,
        grid=(N // RB,),
        in_specs=specs,
        out_specs=pl.BlockSpec((RB, D), lambda i: (i, 0)),
        out_shape=jax.ShapeDtypeStruct((N, D), jnp.float32),
    )(parts, dis, h1, top, W, b, G, fcW, fcb)


def kernel(x, edge_index, top_features, W1, b1, W2, b2, G1, G2, fcW, fcb):
    N, D = x.shape
    E = edge_index.shape[1]
    NJ32 = E // (_NW * _BH)
    RB = 2000

    NJ16 = E // (_NS * _BA)
    src16 = edge_index[0].reshape(_NS, NJ16, _BA)
    dst16 = edge_index[1].reshape(_NS, NJ16, _BA)
    dst32 = edge_index[1].reshape(_NW, NJ32, _BH)

    hist_k = _make_hist(N, NJ32)
    agg_k = _make_agg(N, D, NJ16)

    hist = hist_k(dst32)
    xs1a, xs1b, dis = _prescale(hist, x, RB)
    parts1 = agg_k(src16, dst16, xs1a, xs1b)
    h1, xs2a, xs2b = _mix1(parts1, dis, x, top_features, W1, b1, G1, RB)
    parts2 = agg_k(src16, dst16, xs2a, xs2b)
    return _mix2(parts2, dis, h1, top_features, W2, b2, G2, fcW, fcb, RB)
